# Initial kernel scaffold; baseline (speedup 1.0000x reference)
import jax
import jax.numpy as jnp
from jax.experimental import pallas as pl


def kernel(input, graph):
    raise NotImplementedError("write your pallas kernel here")


# speedup vs baseline: 2.7533x; 2.7533x over previous
import functools

import jax
import jax.numpy as jnp
from jax import lax
from jax.experimental import pallas as pl
from jax.experimental.pallas import tpu as pltpu
from jax.experimental.pallas import tpu_sc as plsc

_N, _D, _S = 32768, 256, 128
_G = 8
_NB = _N // _G
_SB = 256
_SUB = _SB // _G
_LANES = 16
_CH = _D // _LANES
_NC, _NS = 2, 16
_NW = _NC * _NS
_SEGW = _S // _NW


def _prefix_body(x_ref, p8_ref, carry_ref):
    i = pl.program_id(0)

    @pl.when(i == 0)
    def _():
        carry_ref[...] = jnp.zeros_like(carry_ref)

    blk = x_ref[...]
    sub = blk.reshape(_SUB, _G, _D).sum(axis=1)
    ii = lax.broadcasted_iota(jnp.int32, (_SUB, _SUB), 0)
    jj = lax.broadcasted_iota(jnp.int32, (_SUB, _SUB), 1)
    tril = (jj <= ii).astype(jnp.float32)
    inc = jnp.dot(tril, sub, preferred_element_type=jnp.float32)
    inc = inc + carry_ref[...]
    p8_ref[...] = inc
    carry_ref[...] = inc[_SUB - 1:_SUB, :]


def _block_prefix(x):
    return pl.pallas_call(
        _prefix_body,
        grid=(_N // _SB,),
        in_specs=[pl.BlockSpec((_SB, _D), lambda i: (i, 0))],
        out_specs=pl.BlockSpec((_SUB, _D), lambda i: (i, 0)),
        out_shape=jax.ShapeDtypeStruct((_NB, _D), jnp.float32),
        scratch_shapes=[pltpu.VMEM((1, _D), jnp.float32)],
    )(x)


def _sc_body(x_hbm, p8_hbm, starts_hbm, ends_hbm, out_hbm,
             starts_v, ends_v, p8s_v, p8e_v, xs_v, xe_v, outb_v):
    wid = lax.axis_index("s") * _NC + lax.axis_index("c")
    pltpu.sync_copy(starts_hbm, starts_v.at[pl.ds(0, _S)])
    pltpu.sync_copy(ends_hbm, ends_v.at[pl.ds(0, _S)])
    for t in range(_SEGW):
        j = wid * _SEGW + t
        s = starts_v[pl.ds(j, _LANES)][0]
        e = ends_v[pl.ds(j, _LANES)][0]
        be = e + 1
        ks = s // _G
        ke = be // _G
        rs = s - ks * _G
        re = be - ke * _G
        pltpu.sync_copy(p8_hbm.at[pl.ds(jnp.maximum(ks - 1, 0), 1)], p8s_v)
        pltpu.sync_copy(p8_hbm.at[pl.ds(jnp.maximum(ke - 1, 0), 1)], p8e_v)
        pltpu.sync_copy(x_hbm.at[pl.ds(ks * _G, _G)], xs_v)
        pltpu.sync_copy(x_hbm.at[pl.ds(jnp.minimum(ke * _G, _N - _G), _G)],
                        xe_v)
        cntv = jnp.full((_LANES,), (e - s + 1), jnp.int32).astype(jnp.float32)
        inv = 1.0 / cntv
        w_pe = (ke > 0).astype(jnp.float32)
        w_ps = (ks > 0).astype(jnp.float32)
        w_re = [(u < re).astype(jnp.float32) for u in range(_G)]
        w_rs = [(u < rs).astype(jnp.float32) for u in range(_G)]
        for c in range(_CH):
            sl = pl.ds(c * _LANES, _LANES)
            acc = p8e_v[0, sl] * w_pe - p8s_v[0, sl] * w_ps
            for u in range(_G):
                acc = acc + xe_v[u, sl] * w_re[u] - xs_v[u, sl] * w_rs[u]
            outb_v[0, sl] = acc * inv + 0.006
        pltpu.sync_copy(outb_v, out_hbm.at[pl.ds(j, 1)])


def _pool(x, p8, starts, ends):
    mesh = plsc.VectorSubcoreMesh(core_axis_name="c", subcore_axis_name="s")
    return pl.kernel(
        _sc_body,
        out_type=jax.ShapeDtypeStruct((_S, _D), jnp.float32),
        mesh=mesh,
        scratch_types=[
            pltpu.VMEM((_S + _LANES,), jnp.int32),
            pltpu.VMEM((_S + _LANES,), jnp.int32),
            pltpu.VMEM((1, _D), jnp.float32),
            pltpu.VMEM((1, _D), jnp.float32),
            pltpu.VMEM((_G, _D), jnp.float32),
            pltpu.VMEM((_G, _D), jnp.float32),
            pltpu.VMEM((1, _D), jnp.float32),
        ],
    )(x, p8, starts, ends)


@jax.jit
def kernel(input, graph):
    g32 = graph.astype(jnp.int32)
    starts = g32[:, 0]
    ends = g32[:, 1]
    p8 = _block_prefix(input)
    return _pool(input, p8, starts, ends)


# speedup vs baseline: 4.9738x; 1.8065x over previous
import functools

import jax
import jax.numpy as jnp
from jax import lax
from jax.experimental import pallas as pl
from jax.experimental.pallas import tpu as pltpu
from jax.experimental.pallas import tpu_sc as plsc

_N, _D, _S = 32768, 256, 128
_G = 8
_NB = _N // _G
_SB = 1024
_SUB = _SB // _G
_LANES = 16
_CH = _D // _LANES
_NC, _NS = 2, 16
_NW = _NC * _NS
_SEGW = _S // _NW


def _prefix_body(x_ref, p8_ref, carry_ref):
    i = pl.program_id(0)

    @pl.when(i == 0)
    def _():
        carry_ref[...] = jnp.zeros_like(carry_ref)

    blk = x_ref[...]
    sub = blk.reshape(_SUB, _G, _D).sum(axis=1)
    ii = lax.broadcasted_iota(jnp.int32, (_SUB, _SUB), 0)
    jj = lax.broadcasted_iota(jnp.int32, (_SUB, _SUB), 1)
    tril = (jj <= ii).astype(jnp.float32)
    inc = jnp.dot(tril, sub, preferred_element_type=jnp.float32)
    inc = inc + carry_ref[...]
    p8_ref[...] = inc
    carry_ref[...] = inc[_SUB - 1:_SUB, :]


def _block_prefix(x):
    return pl.pallas_call(
        _prefix_body,
        grid=(_N // _SB,),
        in_specs=[pl.BlockSpec((_SB, _D), lambda i: (i, 0))],
        out_specs=pl.BlockSpec((_SUB, _D), lambda i: (i, 0)),
        out_shape=jax.ShapeDtypeStruct((_NB, _D), jnp.float32),
        scratch_shapes=[pltpu.VMEM((1, _D), jnp.float32)],
    )(x)


def _sc_body(x_hbm, p8_hbm, starts_hbm, ends_hbm, out_hbm,
             starts_v, ends_v, p8s_v, p8e_v, xs_v, xe_v, outb_v):
    wid = lax.axis_index("s") * _NC + lax.axis_index("c")
    pltpu.sync_copy(starts_hbm, starts_v.at[pl.ds(0, _S)])
    pltpu.sync_copy(ends_hbm, ends_v.at[pl.ds(0, _S)])
    for t in range(_SEGW):
        j = wid * _SEGW + t
        s = starts_v[pl.ds(j, _LANES)][0]
        e = ends_v[pl.ds(j, _LANES)][0]
        be = e + 1
        ks = s // _G
        ke = be // _G
        rs = s - ks * _G
        re = be - ke * _G
        pltpu.sync_copy(p8_hbm.at[pl.ds(jnp.maximum(ks - 1, 0), 1)], p8s_v)
        pltpu.sync_copy(p8_hbm.at[pl.ds(jnp.maximum(ke - 1, 0), 1)], p8e_v)
        pltpu.sync_copy(x_hbm.at[pl.ds(ks * _G, _G)], xs_v)
        pltpu.sync_copy(x_hbm.at[pl.ds(jnp.minimum(ke * _G, _N - _G), _G)],
                        xe_v)
        cntv = jnp.full((_LANES,), (e - s + 1), jnp.int32).astype(jnp.float32)
        inv = 1.0 / cntv
        w_pe = (ke > 0).astype(jnp.float32)
        w_ps = (ks > 0).astype(jnp.float32)
        w_re = [(u < re).astype(jnp.float32) for u in range(_G)]
        w_rs = [(u < rs).astype(jnp.float32) for u in range(_G)]
        for c in range(_CH):
            sl = pl.ds(c * _LANES, _LANES)
            acc = p8e_v[0, sl] * w_pe - p8s_v[0, sl] * w_ps
            for u in range(_G):
                acc = acc + xe_v[u, sl] * w_re[u] - xs_v[u, sl] * w_rs[u]
            outb_v[0, sl] = acc * inv + 0.006
        pltpu.sync_copy(outb_v, out_hbm.at[pl.ds(j, 1)])


def _pool(x, p8, starts, ends):
    mesh = plsc.VectorSubcoreMesh(core_axis_name="c", subcore_axis_name="s")
    return pl.kernel(
        _sc_body,
        out_type=jax.ShapeDtypeStruct((_S, _D), jnp.float32),
        mesh=mesh,
        scratch_types=[
            pltpu.VMEM((_S + _LANES,), jnp.int32),
            pltpu.VMEM((_S + _LANES,), jnp.int32),
            pltpu.VMEM((1, _D), jnp.float32),
            pltpu.VMEM((1, _D), jnp.float32),
            pltpu.VMEM((_G, _D), jnp.float32),
            pltpu.VMEM((_G, _D), jnp.float32),
            pltpu.VMEM((1, _D), jnp.float32),
        ],
    )(x, p8, starts, ends)


@jax.jit
def kernel(input, graph):
    g32 = graph.astype(jnp.int32)
    starts = g32[:, 0]
    ends = g32[:, 1]
    p8 = _block_prefix(input)
    return _pool(input, p8, starts, ends)


# speedup vs baseline: 6.2266x; 1.2519x over previous
import functools

import jax
import jax.numpy as jnp
from jax import lax
from jax.experimental import pallas as pl
from jax.experimental.pallas import tpu as pltpu
from jax.experimental.pallas import tpu_sc as plsc

_N, _D, _S = 32768, 256, 128
_G = 8
_NB = _N // _G
_SB = 4096
_SUB = _SB // _G
_LANES = 16
_CH = _D // _LANES
_NC, _NS = 2, 16
_NW = _NC * _NS
_SEGW = _S // _NW


def _prefix_body(x_ref, p8_ref, carry_ref):
    i = pl.program_id(0)

    @pl.when(i == 0)
    def _():
        carry_ref[...] = jnp.zeros_like(carry_ref)

    blk = x_ref[...]
    sub = blk.reshape(_SUB, _G, _D).sum(axis=1)
    ii = lax.broadcasted_iota(jnp.int32, (_SUB, _SUB), 0)
    jj = lax.broadcasted_iota(jnp.int32, (_SUB, _SUB), 1)
    tril = (jj <= ii).astype(jnp.float32)
    inc = jnp.dot(tril, sub, preferred_element_type=jnp.float32)
    inc = inc + carry_ref[...]
    p8_ref[...] = inc
    carry_ref[...] = inc[_SUB - 1:_SUB, :]


def _block_prefix(x):
    return pl.pallas_call(
        _prefix_body,
        grid=(_N // _SB,),
        in_specs=[pl.BlockSpec((_SB, _D), lambda i: (i, 0))],
        out_specs=pl.BlockSpec((_SUB, _D), lambda i: (i, 0)),
        out_shape=jax.ShapeDtypeStruct((_NB, _D), jnp.float32),
        scratch_shapes=[pltpu.VMEM((1, _D), jnp.float32)],
    )(x)


def _sc_body(x_hbm, p8_hbm, starts_hbm, ends_hbm, out_hbm,
             starts_v, ends_v, p8s_v, p8e_v, xs_v, xe_v, outb_v):
    wid = lax.axis_index("s") * _NC + lax.axis_index("c")
    pltpu.sync_copy(starts_hbm, starts_v.at[pl.ds(0, _S)])
    pltpu.sync_copy(ends_hbm, ends_v.at[pl.ds(0, _S)])
    for t in range(_SEGW):
        j = wid * _SEGW + t
        s = starts_v[pl.ds(j, _LANES)][0]
        e = ends_v[pl.ds(j, _LANES)][0]
        be = e + 1
        ks = s // _G
        ke = be // _G
        rs = s - ks * _G
        re = be - ke * _G
        pltpu.sync_copy(p8_hbm.at[pl.ds(jnp.maximum(ks - 1, 0), 1)], p8s_v)
        pltpu.sync_copy(p8_hbm.at[pl.ds(jnp.maximum(ke - 1, 0), 1)], p8e_v)
        pltpu.sync_copy(x_hbm.at[pl.ds(ks * _G, _G)], xs_v)
        pltpu.sync_copy(x_hbm.at[pl.ds(jnp.minimum(ke * _G, _N - _G), _G)],
                        xe_v)
        cntv = jnp.full((_LANES,), (e - s + 1), jnp.int32).astype(jnp.float32)
        inv = 1.0 / cntv
        w_pe = (ke > 0).astype(jnp.float32)
        w_ps = (ks > 0).astype(jnp.float32)
        w_re = [(u < re).astype(jnp.float32) for u in range(_G)]
        w_rs = [(u < rs).astype(jnp.float32) for u in range(_G)]
        for c in range(_CH):
            sl = pl.ds(c * _LANES, _LANES)
            acc = p8e_v[0, sl] * w_pe - p8s_v[0, sl] * w_ps
            for u in range(_G):
                acc = acc + xe_v[u, sl] * w_re[u] - xs_v[u, sl] * w_rs[u]
            outb_v[0, sl] = acc * inv + 0.006
        pltpu.sync_copy(outb_v, out_hbm.at[pl.ds(j, 1)])


def _pool(x, p8, starts, ends):
    mesh = plsc.VectorSubcoreMesh(core_axis_name="c", subcore_axis_name="s")
    return pl.kernel(
        _sc_body,
        out_type=jax.ShapeDtypeStruct((_S, _D), jnp.float32),
        mesh=mesh,
        scratch_types=[
            pltpu.VMEM((_S + _LANES,), jnp.int32),
            pltpu.VMEM((_S + _LANES,), jnp.int32),
            pltpu.VMEM((1, _D), jnp.float32),
            pltpu.VMEM((1, _D), jnp.float32),
            pltpu.VMEM((_G, _D), jnp.float32),
            pltpu.VMEM((_G, _D), jnp.float32),
            pltpu.VMEM((1, _D), jnp.float32),
        ],
    )(x, p8, starts, ends)


@jax.jit
def kernel(input, graph):
    g32 = graph.astype(jnp.int32)
    starts = g32[:, 0]
    ends = g32[:, 1]
    p8 = _block_prefix(input)
    return _pool(input, p8, starts, ends)


# speedup vs baseline: 7.2364x; 1.1622x over previous
import functools

import jax
import jax.numpy as jnp
from jax import lax
from jax.experimental import pallas as pl
from jax.experimental.pallas import tpu as pltpu
from jax.experimental.pallas import tpu_sc as plsc

_N, _D, _S = 32768, 256, 128
_G = 8
_NB = _N // _G
_SB = 4096
_SUB = _SB // _G
_LANES = 16
_CH = _D // _LANES
_NC, _NS = 2, 16
_NW = _NC * _NS
_SEGW = _S // _NW


def _prefix_body(x_ref, p8_ref, carry_ref):
    i = pl.program_id(0)

    @pl.when(i == 0)
    def _():
        carry_ref[...] = jnp.zeros_like(carry_ref)

    blk = x_ref[...]
    sub = blk.reshape(_SUB, _G, _D).sum(axis=1)
    ii = lax.broadcasted_iota(jnp.int32, (_SUB, _SUB), 0)
    jj = lax.broadcasted_iota(jnp.int32, (_SUB, _SUB), 1)
    tril = (jj <= ii).astype(jnp.float32)
    inc = jnp.dot(tril, sub, preferred_element_type=jnp.float32)
    inc = inc + carry_ref[...]
    p8_ref[...] = inc
    carry_ref[...] = inc[_SUB - 1:_SUB, :]


def _block_prefix(x):
    return pl.pallas_call(
        _prefix_body,
        grid=(_N // _SB,),
        in_specs=[pl.BlockSpec((_SB, _D), lambda i: (i, 0))],
        out_specs=pl.BlockSpec((_SUB, _D), lambda i: (i, 0)),
        out_shape=jax.ShapeDtypeStruct((_NB, _D), jnp.float32),
        scratch_shapes=[pltpu.VMEM((1, _D), jnp.float32)],
    )(x)


def _sc_body(x_hbm, p8_hbm, starts_hbm, ends_hbm, out_hbm,
             starts_v, ends_v, p8b_v, xb_v, outb_v, sem):
    wid = lax.axis_index("s") * _NC + lax.axis_index("c")
    pltpu.sync_copy(starts_hbm, starts_v.at[pl.ds(0, _S)])
    pltpu.sync_copy(ends_hbm, ends_v.at[pl.ds(0, _S)])
    segs = []
    copies = []
    for t in range(_SEGW):
        j = wid * _SEGW + t
        s = starts_v[pl.ds(j, _LANES)][0]
        e = ends_v[pl.ds(j, _LANES)][0]
        be = e + 1
        ks = s // _G
        ke = be // _G
        segs.append((s, e, ks, ke, be))
        copies.append(pltpu.async_copy(
            p8_hbm.at[pl.ds(jnp.maximum(ks - 1, 0), 1)],
            p8b_v.at[pl.ds(2 * t, 1)], sem))
        copies.append(pltpu.async_copy(
            p8_hbm.at[pl.ds(jnp.maximum(ke - 1, 0), 1)],
            p8b_v.at[pl.ds(2 * t + 1, 1)], sem))
        copies.append(pltpu.async_copy(
            x_hbm.at[pl.ds(ks * _G, _G)],
            xb_v.at[pl.ds(2 * t * _G, _G)], sem))
        copies.append(pltpu.async_copy(
            x_hbm.at[pl.ds(jnp.minimum(ke * _G, _N - _G), _G)],
            xb_v.at[pl.ds((2 * t + 1) * _G, _G)], sem))
    for cpy in copies:
        cpy.wait()
    for t in range(_SEGW):
        s, e, ks, ke, be = segs[t]
        rs = s - ks * _G
        re = be - ke * _G
        cntv = jnp.full((_LANES,), (e - s + 1), jnp.int32).astype(jnp.float32)
        inv = 1.0 / cntv
        w_pe = (ke > 0).astype(jnp.float32)
        w_ps = (ks > 0).astype(jnp.float32)
        w_re = [(u < re).astype(jnp.float32) for u in range(_G)]
        w_rs = [(u < rs).astype(jnp.float32) for u in range(_G)]
        for c in range(_CH):
            sl = pl.ds(c * _LANES, _LANES)
            acc = p8b_v[2 * t + 1, sl] * w_pe - p8b_v[2 * t, sl] * w_ps
            for u in range(_G):
                acc = (acc + xb_v[(2 * t + 1) * _G + u, sl] * w_re[u]
                       - xb_v[2 * t * _G + u, sl] * w_rs[u])
            outb_v[t, sl] = acc * inv + 0.006
    pltpu.sync_copy(outb_v, out_hbm.at[pl.ds(wid * _SEGW, _SEGW)])


def _pool(x, p8, starts, ends):
    mesh = plsc.VectorSubcoreMesh(core_axis_name="c", subcore_axis_name="s")
    return pl.kernel(
        _sc_body,
        out_type=jax.ShapeDtypeStruct((_S, _D), jnp.float32),
        mesh=mesh,
        scratch_types=[
            pltpu.VMEM((_S + _LANES,), jnp.int32),
            pltpu.VMEM((_S + _LANES,), jnp.int32),
            pltpu.VMEM((2 * _SEGW, _D), jnp.float32),
            pltpu.VMEM((2 * _SEGW * _G, _D), jnp.float32),
            pltpu.VMEM((_SEGW, _D), jnp.float32),
            pltpu.SemaphoreType.DMA,
        ],
    )(x, p8, starts, ends)


@jax.jit
def kernel(input, graph):
    g32 = graph.astype(jnp.int32)
    starts = g32[:, 0]
    ends = g32[:, 1]
    p8 = _block_prefix(input)
    return _pool(input, p8, starts, ends)


# speedup vs baseline: 7.4397x; 1.0281x over previous
import functools

import jax
import jax.numpy as jnp
from jax import lax
from jax.experimental import pallas as pl
from jax.experimental.pallas import tpu as pltpu
from jax.experimental.pallas import tpu_sc as plsc

_N, _D, _S = 32768, 256, 128
_G = 8
_NB = _N // _G
_SB = 8192
_SUB = _SB // _G
_LANES = 16
_CH = _D // _LANES
_NC, _NS = 2, 16
_NW = _NC * _NS
_SEGW = _S // _NW


def _prefix_body(x_ref, p8_ref, carry_ref):
    i = pl.program_id(0)

    @pl.when(i == 0)
    def _():
        carry_ref[...] = jnp.zeros_like(carry_ref)

    blk = x_ref[...]
    sub = blk.reshape(_SUB, _G, _D).sum(axis=1)
    ii = lax.broadcasted_iota(jnp.int32, (_SUB, _SUB), 0)
    jj = lax.broadcasted_iota(jnp.int32, (_SUB, _SUB), 1)
    tril = (jj <= ii).astype(jnp.float32)
    inc = jnp.dot(tril, sub, preferred_element_type=jnp.float32)
    inc = inc + carry_ref[...]
    p8_ref[...] = inc
    carry_ref[...] = inc[_SUB - 1:_SUB, :]


def _block_prefix(x):
    return pl.pallas_call(
        _prefix_body,
        grid=(_N // _SB,),
        in_specs=[pl.BlockSpec((_SB, _D), lambda i: (i, 0))],
        out_specs=pl.BlockSpec((_SUB, _D), lambda i: (i, 0)),
        out_shape=jax.ShapeDtypeStruct((_NB, _D), jnp.float32),
        scratch_shapes=[pltpu.VMEM((1, _D), jnp.float32)],
    )(x)


def _sc_body(x_hbm, p8_hbm, starts_hbm, ends_hbm, out_hbm,
             starts_v, ends_v, p8b_v, xb_v, outb_v, sem):
    wid = lax.axis_index("s") * _NC + lax.axis_index("c")
    pltpu.sync_copy(starts_hbm, starts_v.at[pl.ds(0, _S)])
    pltpu.sync_copy(ends_hbm, ends_v.at[pl.ds(0, _S)])
    segs = []
    copies = []
    for t in range(_SEGW):
        j = wid * _SEGW + t
        s = starts_v[pl.ds(j, _LANES)][0]
        e = ends_v[pl.ds(j, _LANES)][0]
        be = e + 1
        ks = s // _G
        ke = be // _G
        segs.append((s, e, ks, ke, be))
        copies.append(pltpu.async_copy(
            p8_hbm.at[pl.ds(jnp.maximum(ks - 1, 0), 1)],
            p8b_v.at[pl.ds(2 * t, 1)], sem))
        copies.append(pltpu.async_copy(
            p8_hbm.at[pl.ds(jnp.maximum(ke - 1, 0), 1)],
            p8b_v.at[pl.ds(2 * t + 1, 1)], sem))
        copies.append(pltpu.async_copy(
            x_hbm.at[pl.ds(ks * _G, _G)],
            xb_v.at[pl.ds(2 * t * _G, _G)], sem))
        copies.append(pltpu.async_copy(
            x_hbm.at[pl.ds(jnp.minimum(ke * _G, _N - _G), _G)],
            xb_v.at[pl.ds((2 * t + 1) * _G, _G)], sem))
    for cpy in copies:
        cpy.wait()
    for t in range(_SEGW):
        s, e, ks, ke, be = segs[t]
        rs = s - ks * _G
        re = be - ke * _G
        cntv = jnp.full((_LANES,), (e - s + 1), jnp.int32).astype(jnp.float32)
        inv = 1.0 / cntv
        w_pe = (ke > 0).astype(jnp.float32)
        w_ps = (ks > 0).astype(jnp.float32)
        w_re = [(u < re).astype(jnp.float32) for u in range(_G)]
        w_rs = [(u < rs).astype(jnp.float32) for u in range(_G)]
        for c in range(_CH):
            sl = pl.ds(c * _LANES, _LANES)
            acc = p8b_v[2 * t + 1, sl] * w_pe - p8b_v[2 * t, sl] * w_ps
            for u in range(_G):
                acc = (acc + xb_v[(2 * t + 1) * _G + u, sl] * w_re[u]
                       - xb_v[2 * t * _G + u, sl] * w_rs[u])
            outb_v[t, sl] = acc * inv + 0.006
    pltpu.sync_copy(outb_v, out_hbm.at[pl.ds(wid * _SEGW, _SEGW)])


def _pool(x, p8, starts, ends):
    mesh = plsc.VectorSubcoreMesh(core_axis_name="c", subcore_axis_name="s")
    return pl.kernel(
        _sc_body,
        out_type=jax.ShapeDtypeStruct((_S, _D), jnp.float32),
        mesh=mesh,
        scratch_types=[
            pltpu.VMEM((_S + _LANES,), jnp.int32),
            pltpu.VMEM((_S + _LANES,), jnp.int32),
            pltpu.VMEM((2 * _SEGW, _D), jnp.float32),
            pltpu.VMEM((2 * _SEGW * _G, _D), jnp.float32),
            pltpu.VMEM((_SEGW, _D), jnp.float32),
            pltpu.SemaphoreType.DMA,
        ],
    )(x, p8, starts, ends)


@jax.jit
def kernel(input, graph):
    g32 = graph.astype(jnp.int32)
    starts = g32[:, 0]
    ends = g32[:, 1]
    p8 = _block_prefix(input)
    return _pool(input, p8, starts, ends)
